# Initial kernel scaffold; baseline (speedup 1.0000x reference)
import jax
import jax.numpy as jnp
from jax.experimental import pallas as pl


def kernel(user_embed, item_embed, adj_values, adj_indices):
    raise NotImplementedError("write your pallas kernel here")


# speedup vs baseline: 7.4798x; 7.4798x over previous
import jax
import jax.numpy as jnp
from jax import lax
from jax.experimental import pallas as pl
from jax.experimental.pallas import tpu as pltpu
from jax.experimental.pallas import tpu_sc as plsc

N_USERS = 20000
N_ITEMS = 30000
N_NODES = N_USERS + N_ITEMS
N_EDGES = 800000
EMB_DIM = 64
HALF = 32
N_HOPS = 3

NC = 2
NS = 16
LANES = 16

CHUNK = 128
EPT = -(-N_EDGES // (NS * CHUNK)) * CHUNK
E_PAD = NS * EPT
N_CHUNKS = EPT // CHUNK

N_PAD = 51200
RPT = N_PAD // NS
DRAIN = 128


def _bcast_lane(v, e):
    idx = jnp.full((LANES, 1), e, jnp.int32)
    dn = lax.GatherDimensionNumbers(
        offset_dims=(), collapsed_slice_dims=(0,), start_index_map=(0,))
    return lax.gather(v, idx, dn, (1,),
                      mode=lax.GatherScatterMode.PROMISE_IN_BOUNDS)


def _gcn_body(edges_hbm, ego_hbm,
              mean_hbm, l1_hbm, e2_hbm,
              acc, ebuf, gbuf, dbuf, l1b, e2b, zbuf, gsem, ssem):
    cid = lax.axis_index("c")
    sid = lax.axis_index("s")
    r_base = sid * RPT

    zv = jnp.zeros((LANES,), jnp.float32)

    def _zero_row(r, carry):
        zbuf[r, pl.ds(0, LANES)] = zv
        zbuf[r, pl.ds(LANES, LANES)] = zv
        return carry

    lax.fori_loop(0, DRAIN, _zero_row, 0)
    for j in range(RPT // DRAIN):
        pltpu.sync_copy(zbuf, acc.at[pl.ds(r_base + j * DRAIN, DRAIN)])
    plsc.subcore_barrier()

    base_off = cid * N_PAD

    def _load_edges(ci, q):
        pltpu.sync_copy(edges_hbm.at[sid * N_CHUNKS + ci], ebuf.at[q])
        for g in range(CHUNK // LANES):
            sl = pl.ds(g * LANES, LANES)
            ebuf[q, 0, sl] = ebuf[q, 0, sl] + base_off

    def _scale(p):
        for g in range(CHUNK // LANES):
            vv = lax.bitcast_convert_type(
                ebuf[p, 2, pl.ds(g * LANES, LANES)], jnp.float32)
            for e in range(LANES):
                vb = _bcast_lane(vv, e)
                row = g * LANES + e
                gbuf[p, row, pl.ds(0, LANES)] = (
                    gbuf[p, row, pl.ds(0, LANES)] * vb)
                gbuf[p, row, pl.ds(LANES, LANES)] = (
                    gbuf[p, row, pl.ds(LANES, LANES)] * vb)

    for k in range(N_HOPS):
        src = (ego_hbm, l1_hbm, e2_hbm)[k]

        def _gather_start(q, src=src):
            pltpu.async_copy(src.at[ebuf.at[q, 0]], gbuf.at[q], gsem.at[q])

        def _gather_wait(p, src=src):
            pltpu.make_async_copy(
                src.at[ebuf.at[p, 0]], gbuf.at[p], gsem.at[p]).wait()

        def _scatter_start(p):
            pltpu.async_copy(gbuf.at[p], acc.at[ebuf.at[p, 1]], ssem.at[p],
                             add=True)

        def _scatter_wait(q):
            pltpu.make_async_copy(
                gbuf.at[q], acc.at[ebuf.at[q, 1]], ssem.at[q]).wait()

        _load_edges(0, 0)
        _gather_start(0)

        def _chunk(ci, carry):
            p = jnp.bitwise_and(ci, 1)
            q = 1 - p

            @pl.when(ci + 1 < N_CHUNKS)
            def _():
                @pl.when(ci >= 1)
                def _():
                    _scatter_wait(q)
                _load_edges(ci + 1, q)
                _gather_start(q)

            @pl.when(ci + 1 == N_CHUNKS)
            def _():
                _scatter_wait(q)

            _gather_wait(p)
            _scale(p)
            _scatter_start(p)
            return carry

        lax.fori_loop(0, N_CHUNKS, _chunk, 0)
        _scatter_wait((N_CHUNKS - 1) % 2)
        plsc.subcore_barrier()

        for j in range(RPT // DRAIN):
            r0 = r_base + j * DRAIN
            dst = cid * N_PAD + r0
            pltpu.sync_copy(acc.at[pl.ds(r0, DRAIN)], dbuf)
            if k < 2:
                pltpu.sync_copy(zbuf, acc.at[pl.ds(r0, DRAIN)])
            if k == 0:
                pltpu.sync_copy(dbuf, l1_hbm.at[pl.ds(dst, DRAIN)])
            elif k == 1:
                pltpu.sync_copy(dbuf, e2_hbm.at[pl.ds(dst, DRAIN)])
            else:
                pltpu.sync_copy(l1_hbm.at[pl.ds(dst, DRAIN)], l1b)
                pltpu.sync_copy(e2_hbm.at[pl.ds(dst, DRAIN)], e2b)

                def _mean_row(r, carry):
                    for h in range(2):
                        sl = pl.ds(h * LANES, LANES)
                        dbuf[r, sl] = (
                            dbuf[r, sl] + l1b[r, sl] + e2b[r, sl]
                        ) * jnp.float32(1.0 / 3.0)
                    return carry

                lax.fori_loop(0, DRAIN, _mean_row, 0)
                pltpu.sync_copy(dbuf, mean_hbm.at[pl.ds(dst, DRAIN)])
        plsc.subcore_barrier()


_gcn = pl.kernel(
    _gcn_body,
    out_type=[
        jax.ShapeDtypeStruct((2 * N_PAD, HALF), jnp.float32),
        jax.ShapeDtypeStruct((2 * N_PAD, HALF), jnp.float32),
        jax.ShapeDtypeStruct((2 * N_PAD, HALF), jnp.float32),
    ],
    mesh=plsc.VectorSubcoreMesh(core_axis_name="c", subcore_axis_name="s"),
    compiler_params=pltpu.CompilerParams(use_tc_tiling_on_sc=False),
    scratch_types=[
        pltpu.VMEM_SHARED((N_PAD, HALF), jnp.float32),
        pltpu.VMEM((2, 3, CHUNK), jnp.int32),
        pltpu.VMEM((2, CHUNK, HALF), jnp.float32),
        pltpu.VMEM((DRAIN, HALF), jnp.float32),
        pltpu.VMEM((DRAIN, HALF), jnp.float32),
        pltpu.VMEM((DRAIN, HALF), jnp.float32),
        pltpu.VMEM((DRAIN, HALF), jnp.float32),
        pltpu.SemaphoreType.DMA((2,)),
        pltpu.SemaphoreType.DMA((2,)),
    ],
)


def kernel(user_embed, item_embed, adj_values, adj_indices):
    ego0 = jnp.concatenate([user_embed, item_embed], axis=0)
    ego_p = jnp.pad(ego0, ((0, N_PAD - N_NODES), (0, 0)))
    ego_s = (ego_p.reshape(N_PAD, NC, HALF)
             .transpose(1, 0, 2)
             .reshape(NC * N_PAD, HALF))
    rows = adj_indices[0]
    cols = adj_indices[1]
    pad = E_PAD - N_EDGES
    n_rows = E_PAD // CHUNK
    rows_p = jnp.pad(rows, (0, pad)).reshape(n_rows, CHUNK)
    cols_p = jnp.pad(cols, (0, pad)).reshape(n_rows, CHUNK)
    vals_i = lax.bitcast_convert_type(
        jnp.pad(adj_values, (0, pad)), jnp.int32).reshape(n_rows, CHUNK)
    edges = jnp.stack([cols_p, rows_p, vals_i], axis=1)

    mean_s, l1_s, _ = _gcn(edges, ego_s)

    def unstack(x):
        return (x.reshape(NC, N_PAD, HALF)[:, :N_NODES]
                .transpose(1, 0, 2)
                .reshape(N_NODES, EMB_DIM))

    mean = unstack(mean_s)
    l1 = unstack(l1_s)
    return (mean[:N_USERS], mean[N_USERS:], l1[:N_USERS], l1[N_USERS:])
